# baseline (speedup 1.0000x reference)
import functools

import jax
import jax.numpy as jnp
import numpy as np
from jax import lax
from jax.experimental import pallas as pl
from jax.experimental.pallas import tpu as pltpu
from jax.experimental.pallas import tpu_sc as plsc

N = 10000
NP = 10240
E = 320000
F_IN = 128
NH = 64
LW = 16
NC = 2
NS = 16
NW = NC * NS
EPT = E // NW
BR = NP // NW
CAP = 10880
CH = 128
K1ROWS = NW * CAP // CH // NW
GRP = 80
_f32 = jnp.float32
_i32 = jnp.int32

_SC_FLAT = pltpu.CompilerParams(use_tc_tiling_on_sc=False,
                                needs_layout_passes=False)
_SC_STD = pltpu.CompilerParams(use_tc_tiling_on_sc=False)
_DN = lax.GatherDimensionNumbers(
    offset_dims=(), collapsed_slice_dims=(0,), start_index_map=(0,))


def _mesh():
    return plsc.VectorSubcoreMesh(core_axis_name="c", subcore_axis_name="s")


def _bcast_last(v):
    return lax.gather(v, jnp.full((LW, 1), LW - 1, _i32), _DN, (1,),
                      mode=lax.GatherScatterMode.PROMISE_IN_BOUNDS)


def _lane_splat(v, r):
    return lax.gather(v, jnp.full((LW, 1), r, _i32), _DN, (1,),
                      mode=lax.GatherScatterMode.PROMISE_IN_BOUNDS)


def _sc_count(dstv):
    @functools.partial(
        pl.kernel, mesh=_mesh(), compiler_params=_SC_FLAT,
        out_type=jax.ShapeDtypeStruct((NW * NW * LW,), _i32),
        scratch_types=[
            pltpu.VMEM((EPT,), _i32),
            pltpu.VMEM((NW * LW,), _i32),
        ],
    )
    def k(d_hbm, c_hbm, dbuf, cnt):
        wid = lax.axis_index("c") * NS + lax.axis_index("s")
        iot = lax.iota(_i32, LW)
        pltpu.sync_copy(d_hbm.at[pl.ds(wid * EPT, EPT)], dbuf)

        @pl.loop(0, NW)
        def _(j):
            cnt[pl.ds(j * LW, LW)] = jnp.zeros((LW,), _i32)

        @pl.loop(0, EPT // LW)
        def _(r):
            d = dbuf[pl.ds(r * LW, LW)]
            w = (d * 6554) >> 21
            addr = w * LW + iot
            c = plsc.load_gather(cnt, [addr])
            plsc.store_scatter(cnt, [addr], c + 1)

        pltpu.sync_copy(cnt, c_hbm.at[pl.ds(wid * NW * LW, NW * LW)])

    return k(dstv)


def _sc_place(srcv, dstv, counts):
    @functools.partial(
        pl.kernel, mesh=_mesh(), compiler_params=_SC_FLAT,
        out_type=[jax.ShapeDtypeStruct((NW * CAP,), _i32),
                  jax.ShapeDtypeStruct((NW * CAP,), _i32)],
        scratch_types=[
            pltpu.VMEM((EPT,), _i32),
            pltpu.VMEM((EPT,), _i32),
            pltpu.VMEM((NW * NW * LW,), _i32),
            pltpu.VMEM((NW * LW,), _i32),
            pltpu.VMEM((GRP,), _i32),
            pltpu.VMEM((GRP,), _i32),
            pltpu.SemaphoreType.DMA,
            pltpu.SemaphoreType.DMA,
        ],
    )
    def k(s_hbm, d_hbm, c_hbm, sb_hbm, db_hbm, sbuf, dbuf, cnts, offt,
          pos0, pos1, sem0, sem1):
        wid = lax.axis_index("c") * NS + lax.axis_index("s")
        iot = lax.iota(_i32, LW)
        poss = (pos0, pos1)
        sems = (sem0, sem1)
        pltpu.sync_copy(s_hbm.at[pl.ds(wid * EPT, EPT)], sbuf)
        pltpu.sync_copy(d_hbm.at[pl.ds(wid * EPT, EPT)], dbuf)
        pltpu.sync_copy(c_hbm, cnts)

        for w in range(NW):
            prev = jnp.zeros((LW,), _i32)
            for t2 in range(NW):
                v = cnts[pl.ds((t2 * NW + w) * LW, LW)]
                prev = prev + jnp.where(t2 < wid, v, 0)
            own = cnts[pl.ds((wid * NW + w) * LW, LW)]
            sprev = _bcast_last(plsc.cumsum(prev))
            excl = plsc.cumsum(own) - own
            offt[pl.ds(w * LW, LW)] = w * CAP + sprev + excl

        def flush(f, b):
            pltpu.async_copy(sbuf.at[pl.ds(f * GRP, GRP)],
                             sb_hbm.at[poss[b]], sems[b])
            pltpu.async_copy(dbuf.at[pl.ds(f * GRP, GRP)],
                             db_hbm.at[poss[b]], sems[b])

        def wait_flush(f, b):
            pltpu.make_async_copy(sbuf.at[pl.ds(f * GRP, GRP)],
                                  sb_hbm.at[poss[b]], sems[b]).wait()
            pltpu.make_async_copy(dbuf.at[pl.ds(f * GRP, GRP)],
                                  db_hbm.at[poss[b]], sems[b]).wait()

        @pl.loop(0, EPT // GRP)
        def _(f):
            b0 = f % 2

            @pl.when(f >= 2)
            def _():
                for b in range(2):
                    @pl.when(b0 == b)
                    def _():
                        wait_flush(f - 2, b)

            @pl.loop(0, GRP // LW)
            def _(j):
                r = f * (GRP // LW) + j
                d = dbuf[pl.ds(r * LW, LW)]
                w = (d * 6554) >> 21
                addr = w * LW + iot
                p = plsc.load_gather(offt, [addr])
                plsc.store_scatter(offt, [addr], p + 1)
                for b in range(2):
                    @pl.when(b0 == b)
                    def _():
                        poss[b][pl.ds(j * LW, LW)] = p

            for b in range(2):
                @pl.when(b0 == b)
                def _():
                    flush(f, b)

        wait_flush(EPT // GRP - 2, (EPT // GRP - 2) % 2)
        wait_flush(EPT // GRP - 1, (EPT // GRP - 1) % 2)

    return k(srcv, dstv, counts)


def _sc_compute(srcb2d, dstb2d, s_tab, d_tab, cvec_hbm, h_tab, heads8):
    @functools.partial(
        pl.kernel, mesh=_mesh(), compiler_params=_SC_STD,
        out_type=[
            jax.ShapeDtypeStruct((NW * CAP, LW), _f32),
            jax.ShapeDtypeStruct((NW * CAP, NH), _f32),
        ],
        scratch_types=[
            pltpu.VMEM((K1ROWS, CH), _i32),
            pltpu.VMEM((K1ROWS, CH), _i32),
            pltpu.VMEM((CH, LW), _f32),
            pltpu.VMEM((CH, LW), _f32),
            pltpu.VMEM((CH, LW), _f32),
            pltpu.VMEM((CH, LW), _f32),
            pltpu.VMEM((CH, NH), _f32),
            pltpu.VMEM((CH, NH), _f32),
            pltpu.VMEM((CH, LW), _f32),
            pltpu.VMEM((CH, LW), _f32),
            pltpu.VMEM((CH, NH), _f32),
            pltpu.VMEM((CH, NH), _f32),
            pltpu.VMEM((LW,), _f32),
            pltpu.SemaphoreType.DMA,
            pltpu.SemaphoreType.DMA,
            pltpu.SemaphoreType.DMA,
            pltpu.SemaphoreType.DMA,
        ],
    )
    def k(src_hbm, dst_hbm, s_hbm, d_hbm, c_hbm, h_hbm, ex_hbm, mg_hbm,
          sidx, didx, srow0, srow1, drow0, drow1, hrow0, hrow1,
          exb0, exb1, msg0, msg1, cvec, semg0, semg1, semw0, semw1):
        wid = lax.axis_index("c") * NS + lax.axis_index("s")
        srow = (srow0, srow1)
        drow = (drow0, drow1)
        hrow = (hrow0, hrow1)
        exb = (exb0, exb1)
        msg = (msg0, msg1)
        semg = (semg0, semg1)
        semw = (semw0, semw1)
        r0 = wid * K1ROWS
        pltpu.sync_copy(src_hbm.at[pl.ds(r0, K1ROWS)], sidx)
        pltpu.sync_copy(dst_hbm.at[pl.ds(r0, K1ROWS)], didx)
        pltpu.sync_copy(c_hbm, cvec)
        cv = cvec[...]
        iot = lax.iota(_i32, LW)
        repidx = (iot % 8 if heads8 else iot * 0).reshape(LW, 1)

        @pl.loop(0, K1ROWS)
        def _(i):
            for j in range(CH // LW):
                s = sidx[i, pl.ds(j * LW, LW)]
                sidx[i, pl.ds(j * LW, LW)] = jnp.clip(s, 0, N - 1)
                d = didx[i, pl.ds(j * LW, LW)]
                didx[i, pl.ds(j * LW, LW)] = jnp.clip(d, 0, N - 1)

        def start_gathers(i, b):
            pltpu.async_copy(s_hbm.at[sidx.at[i]], srow[b], semg[b])
            pltpu.async_copy(d_hbm.at[didx.at[i]], drow[b], semg[b])
            pltpu.async_copy(h_hbm.at[sidx.at[i]], hrow[b], semg[b])

        def wait_gathers(i, b):
            pltpu.make_async_copy(s_hbm.at[sidx.at[i]], srow[b], semg[b]).wait()
            pltpu.make_async_copy(d_hbm.at[didx.at[i]], drow[b], semg[b]).wait()
            pltpu.make_async_copy(h_hbm.at[sidx.at[i]], hrow[b], semg[b]).wait()

        def start_writes(i, b):
            sl = pl.ds((r0 + i) * CH, CH)
            pltpu.async_copy(exb[b], ex_hbm.at[sl], semw[b])
            pltpu.async_copy(msg[b], mg_hbm.at[sl], semw[b])

        def wait_writes(i, b):
            sl = pl.ds((r0 + i) * CH, CH)
            pltpu.make_async_copy(exb[b], ex_hbm.at[sl], semw[b]).wait()
            pltpu.make_async_copy(msg[b], mg_hbm.at[sl], semw[b]).wait()

        start_gathers(0, 0)
        start_gathers(1, 1)

        @pl.loop(0, K1ROWS // 2 + 1)
        def _(t):
            for b in range(2):
                i = t * 2 + b

                @pl.when(i < K1ROWS)
                def _():
                    wait_gathers(i, b)

                    @pl.when(i >= 2)
                    def _():
                        wait_writes(i - 2, b)

                    @pl.loop(0, CH)
                    def _(r):
                        v = srow[b][r, :] + drow[b][r, :]
                        a = jnp.where(v >= 0.0, v, 0.2 * v)
                        e = jnp.exp(a - cv)
                        exb[b][r, :] = e
                        rep = lax.gather(
                            e, repidx, _DN, (1,),
                            mode=lax.GatherScatterMode.PROMISE_IN_BOUNDS)
                        for j in range(NH // LW):
                            msg[b][r, pl.ds(j * LW, LW)] = (
                                hrow[b][r, pl.ds(j * LW, LW)] * rep)

                    start_writes(i, b)

                    @pl.when(i + 2 < K1ROWS)
                    def _():
                        start_gathers(i + 2, b)

        wait_writes(K1ROWS - 2, (K1ROWS - 2) % 2)
        wait_writes(K1ROWS - 1, (K1ROWS - 1) % 2)

    return k(srcb2d, dstb2d, s_tab, d_tab, cvec_hbm, h_tab)


def _sc_accum(dstb, counts, exh_flat, mgh_flat):
    TR_A = BR * NH
    TR_D = BR * LW

    @functools.partial(
        pl.kernel, mesh=_mesh(), compiler_params=_SC_FLAT,
        out_type=[jax.ShapeDtypeStruct((NP * LW,), _f32),
                  jax.ShapeDtypeStruct((NP * NH,), _f32)],
        scratch_types=[
            pltpu.VMEM((NW * NW * LW,), _i32),
            pltpu.VMEM((BR * NH + LW,), _f32),
            pltpu.VMEM((BR * LW + LW,), _f32),
            pltpu.VMEM((CH,), _i32),
            pltpu.VMEM((CH,), _i32),
            pltpu.VMEM((CH * LW,), _f32),
            pltpu.VMEM((CH * LW,), _f32),
            pltpu.VMEM((CH * NH,), _f32),
            pltpu.VMEM((CH * NH,), _f32),
            pltpu.SemaphoreType.DMA,
            pltpu.SemaphoreType.DMA,
        ],
    )
    def k(db_hbm, c_hbm, ex_hbm, mg_hbm, den_hbm, out_hbm,
          cnts, acc, den, dbuf0, dbuf1, exb0, exb1, mgb0, mgb1, sem0, sem1):
        wid = lax.axis_index("c") * NS + lax.axis_index("s")
        iot = lax.iota(_i32, LW)
        dbuf = (dbuf0, dbuf1)
        exbf = (exb0, exb1)
        mgbf = (mgb0, mgb1)
        sems = (sem0, sem1)
        pltpu.sync_copy(c_hbm, cnts)

        totv = jnp.zeros((LW,), _i32)
        for t2 in range(NW):
            totv = totv + cnts[pl.ds((t2 * NW) * LW + wid * LW, LW)]
        cnt_spl = _bcast_last(plsc.cumsum(totv))

        @pl.loop(0, (BR * NH + LW) // LW)
        def _(j):
            acc[pl.ds(j * LW, LW)] = jnp.zeros((LW,), _f32)

        @pl.loop(0, (BR * LW + LW) // LW)
        def _(j):
            den[pl.ds(j * LW, LW)] = jnp.zeros((LW,), _f32)

        e0 = wid * CAP

        def start_reads(c, b):
            pltpu.async_copy(db_hbm.at[pl.ds(e0 + c * CH, CH)], dbuf[b], sems[b])
            pltpu.async_copy(ex_hbm.at[pl.ds((e0 + c * CH) * LW, CH * LW)],
                             exbf[b], sems[b])
            pltpu.async_copy(mg_hbm.at[pl.ds((e0 + c * CH) * NH, CH * NH)],
                             mgbf[b], sems[b])

        def wait_reads(c, b):
            pltpu.make_async_copy(db_hbm.at[pl.ds(e0 + c * CH, CH)],
                                  dbuf[b], sems[b]).wait()
            pltpu.make_async_copy(ex_hbm.at[pl.ds((e0 + c * CH) * LW, CH * LW)],
                                  exbf[b], sems[b]).wait()
            pltpu.make_async_copy(mg_hbm.at[pl.ds((e0 + c * CH) * NH, CH * NH)],
                                  mgbf[b], sems[b]).wait()

        start_reads(0, 0)
        start_reads(1, 1)

        @pl.loop(0, K1ROWS // 2 + 1)
        def _(t):
            for b in range(2):
                c = t * 2 + b

                @pl.when(c < K1ROWS)
                def _():
                    wait_reads(c, b)

                    @pl.loop(0, CH // LW)
                    def _(j):
                        dl = dbuf[b][pl.ds(j * LW, LW)] - wid * BR
                        for r2 in range(LW):
                            dspl = _lane_splat(dl, r2)
                            gidx = jax.lax.broadcast(
                                c * CH + j * LW + r2, (LW,))
                            valid = gidx < cnt_spl
                            ei = j * LW + r2
                            da = jnp.where(valid, dspl * LW + iot, TR_D + iot)
                            plsc.addupdate_scatter(
                                den, [da], exbf[b][pl.ds(ei * LW, LW)])
                            ab = dspl * NH
                            for g in range(NH // LW):
                                aa = jnp.where(valid, ab + g * LW + iot,
                                               TR_A + iot)
                                plsc.addupdate_scatter(
                                    acc, [aa],
                                    mgbf[b][pl.ds(ei * NH + g * LW, LW)])

                    @pl.when(c + 2 < K1ROWS)
                    def _():
                        start_reads(c + 2, b)

        pltpu.sync_copy(den.at[pl.ds(0, BR * LW)],
                        den_hbm.at[pl.ds(wid * BR * LW, BR * LW)])
        pltpu.sync_copy(acc.at[pl.ds(0, BR * NH)],
                        out_hbm.at[pl.ds(wid * BR * NH, BR * NH)])

    return k(dstb, counts, exh_flat, mgh_flat)


def _tc_pre1(x, w1p, asrc_p, adst_p):
    blk = 1000

    def body(x_ref, w_ref, as_ref, ad_ref, h_ref, s_ref, d_ref, c_ref, mx_ref):
        i = pl.program_id(0)
        h = jnp.dot(x_ref[...], w_ref[...], preferred_element_type=_f32)
        h_ref[...] = h
        s = jnp.dot(h, as_ref[...], preferred_element_type=_f32)
        d = jnp.dot(h, ad_ref[...], preferred_element_type=_f32)
        s_ref[...] = s
        d_ref[...] = d
        m = jnp.concatenate([jnp.max(s, axis=0, keepdims=True),
                             jnp.max(d, axis=0, keepdims=True)], axis=0)

        @pl.when(i == 0)
        def _():
            mx_ref[...] = m

        @pl.when(i > 0)
        def _():
            mx_ref[...] = jnp.maximum(mx_ref[...], m)

        c_ref[...] = mx_ref[0:1] + mx_ref[1:2]

    return pl.pallas_call(
        body,
        grid=(N // blk,),
        in_specs=[
            pl.BlockSpec((blk, F_IN), lambda i: (i, 0)),
            pl.BlockSpec((F_IN, NH), lambda i: (0, 0)),
            pl.BlockSpec((NH, LW), lambda i: (0, 0)),
            pl.BlockSpec((NH, LW), lambda i: (0, 0)),
        ],
        out_specs=[
            pl.BlockSpec((blk, NH), lambda i: (i, 0)),
            pl.BlockSpec((blk, LW), lambda i: (i, 0)),
            pl.BlockSpec((blk, LW), lambda i: (i, 0)),
            pl.BlockSpec((1, LW), lambda i: (0, 0)),
        ],
        out_shape=[
            jax.ShapeDtypeStruct((N, NH), _f32),
            jax.ShapeDtypeStruct((N, LW), _f32),
            jax.ShapeDtypeStruct((N, LW), _f32),
            jax.ShapeDtypeStruct((1, LW), _f32),
        ],
        scratch_shapes=[pltpu.VMEM((2, LW), _f32)],
    )(x, w1p, asrc_p, adst_p)


def _tc_mid(out1, den1, b1p, g1p, be1p, w2p, as2p, ad2p):
    blk = 1024
    ibn = 1.0 / np.sqrt(1.0 + 1e-5)

    def body(o_ref, dp_ref, b_ref, g_ref, be_ref, w_ref, as_ref, ad_ref,
             h_ref, s_ref, d_ref, c_ref, mx_ref):
        i = pl.program_id(0)
        rec = 1.0 / (dp_ref[...] + 1e-16)
        rec_rep = jnp.concatenate([rec[:, 0:8]] * 8, axis=1)
        z = o_ref[...] * rec_rep + b_ref[...]
        z = jax.nn.relu(z * ibn * g_ref[...] + be_ref[...])
        h = jnp.dot(z, w_ref[...], preferred_element_type=_f32)
        h_ref[...] = h
        s = jnp.dot(h, as_ref[...], preferred_element_type=_f32)
        d = jnp.dot(h, ad_ref[...], preferred_element_type=_f32)
        s_ref[...] = s
        d_ref[...] = d
        m = jnp.concatenate([jnp.max(s, axis=0, keepdims=True),
                             jnp.max(d, axis=0, keepdims=True)], axis=0)

        @pl.when(i == 0)
        def _():
            mx_ref[...] = m

        @pl.when(i > 0)
        def _():
            mx_ref[...] = jnp.maximum(mx_ref[...], m)

        c_ref[...] = mx_ref[0:1] + mx_ref[1:2]

    return pl.pallas_call(
        body,
        grid=(NP // blk,),
        in_specs=[
            pl.BlockSpec((blk, NH), lambda i: (i, 0)),
            pl.BlockSpec((blk, LW), lambda i: (i, 0)),
            pl.BlockSpec((1, NH), lambda i: (0, 0)),
            pl.BlockSpec((1, NH), lambda i: (0, 0)),
            pl.BlockSpec((1, NH), lambda i: (0, 0)),
            pl.BlockSpec((NH, NH), lambda i: (0, 0)),
            pl.BlockSpec((NH, LW), lambda i: (0, 0)),
            pl.BlockSpec((NH, LW), lambda i: (0, 0)),
        ],
        out_specs=[
            pl.BlockSpec((blk, NH), lambda i: (i, 0)),
            pl.BlockSpec((blk, LW), lambda i: (i, 0)),
            pl.BlockSpec((blk, LW), lambda i: (i, 0)),
            pl.BlockSpec((1, LW), lambda i: (0, 0)),
        ],
        out_shape=[
            jax.ShapeDtypeStruct((NP, NH), _f32),
            jax.ShapeDtypeStruct((NP, LW), _f32),
            jax.ShapeDtypeStruct((NP, LW), _f32),
            jax.ShapeDtypeStruct((1, LW), _f32),
        ],
        scratch_shapes=[pltpu.VMEM((2, LW), _f32)],
    )(out1, den1, b1p, g1p, be1p, w2p, as2p, ad2p)


def _tc_head(out2, den2, b2, cg, cb, wl1, bl1, g1, be1, wl2, bl2, g2, be2,
             wf, bf):
    blk = 1024
    ibn = 1.0 / np.sqrt(1.0 + 1e-5)

    def body(o_ref, dp_ref, b2_ref, cg_ref, cb_ref, w1_ref, b1_ref, g1_ref,
             be1_ref, w2_ref, b2b_ref, g2_ref, be2_ref, wf_ref, bf_ref, y_ref):
        rec = 1.0 / (dp_ref[:, 0:1] + 1e-16)
        g = o_ref[...] * rec + b2_ref[...]
        g = g * ibn * cg_ref[...] + cb_ref[...]
        t = jnp.dot(g, w1_ref[...], preferred_element_type=_f32) + b1_ref[...]
        t = jax.nn.relu(t * ibn * g1_ref[...] + be1_ref[...])
        t = jnp.dot(t, w2_ref[...], preferred_element_type=_f32) + b2b_ref[...]
        t = jax.nn.relu(t * ibn * g2_ref[...] + be2_ref[...])
        y = jnp.dot(t, wf_ref[...], preferred_element_type=_f32) + bf_ref[...]
        y_ref[...] = jax.nn.sigmoid(y)

    vec = lambda: pl.BlockSpec((1, NH), lambda i: (0, 0))
    return pl.pallas_call(
        body,
        grid=(NP // blk,),
        in_specs=[
            pl.BlockSpec((blk, NH), lambda i: (i, 0)),
            pl.BlockSpec((blk, LW), lambda i: (i, 0)),
            vec(), vec(), vec(),
            pl.BlockSpec((NH, NH), lambda i: (0, 0)),
            vec(), vec(), vec(),
            pl.BlockSpec((NH, NH), lambda i: (0, 0)),
            vec(), vec(), vec(),
            pl.BlockSpec((NH, 1), lambda i: (0, 0)),
            pl.BlockSpec((1, 1), lambda i: (0, 0)),
        ],
        out_specs=pl.BlockSpec((blk, 1), lambda i: (i, 0)),
        out_shape=jax.ShapeDtypeStruct((NP, 1), _f32),
    )(out2, den2, b2, cg, cb, wl1, bl1, g1, be1, wl2, bl2, g2, be2, wf, bf)


def _sc_layer(srcb2d, dstb2d, dstb, counts, s_tab, d_tab, cvec, h_tab, heads8):
    exh, mgh = _sc_compute(srcb2d, dstb2d, s_tab, d_tab, cvec, h_tab, heads8)
    den_f, out_f = _sc_accum(dstb, counts, exh.reshape(NW * CAP * LW),
                             mgh.reshape(NW * CAP * NH))
    return den_f.reshape(NP, LW), out_f.reshape(NP, NH)


def kernel(x, edge_index, W1, a_src1, a_dst1, b1, W2, a_src2, a_dst2, b2,
           bn_c1_g, bn_c1_b, bn_c2_g, bn_c2_b, Wl1, bl1, bn1_g, bn1_b,
           Wl2, bl2, bn2_g, bn2_b, Wf, bf):
    srcv = edge_index[0]
    dstv = edge_index[1]

    counts = _sc_count(dstv)
    srcb, dstb = _sc_place(srcv, dstv, counts)
    srcb2d = srcb.reshape(NW * CAP // CH, CH)
    dstb2d = dstb.reshape(NW * CAP // CH, CH)

    perm = np.array([(j % 8) * 8 + j // 8 for j in range(NH)])
    w1p = W1[:, perm]
    eye8 = jnp.eye(8, dtype=_f32)
    asrc_p = jnp.concatenate(
        [(a_src1.T[:, :, None] * eye8[None]).reshape(NH, 8),
         jnp.zeros((NH, 8), _f32)], axis=1)
    adst_p = jnp.concatenate(
        [(a_dst1.T[:, :, None] * eye8[None]).reshape(NH, 8),
         jnp.zeros((NH, 8), _f32)], axis=1)

    h1p, s1, d1, c1 = _tc_pre1(x, w1p, asrc_p, adst_p)
    den1, out1 = _sc_layer(srcb2d, dstb2d, dstb, counts, s1, d1,
                           c1.reshape(LW), h1p, heads8=True)

    w2p = W2[perm, :]
    as2p = jnp.concatenate([a_src2.T, jnp.zeros((NH, LW - 1), _f32)], axis=1)
    ad2p = jnp.concatenate([a_dst2.T, jnp.zeros((NH, LW - 1), _f32)], axis=1)
    h2, s2, d2, c2 = _tc_mid(
        out1, den1, b1[perm].reshape(1, NH), bn_c1_g[perm].reshape(1, NH),
        bn_c1_b[perm].reshape(1, NH), w2p, as2p, ad2p)
    den2, out2 = _sc_layer(srcb2d, dstb2d, dstb, counts, s2, d2,
                           c2.reshape(LW), h2, heads8=False)

    y = _tc_head(
        out2, den2, b2.reshape(1, NH), bn_c2_g.reshape(1, NH),
        bn_c2_b.reshape(1, NH), Wl1, bl1.reshape(1, NH),
        bn1_g.reshape(1, NH), bn1_b.reshape(1, NH), Wl2,
        bl2.reshape(1, NH), bn2_g.reshape(1, NH), bn2_b.reshape(1, NH),
        Wf, bf.reshape(1, 1))
    return y[:N]


# speedup vs baseline: 2.9614x; 2.9614x over previous
import functools

import jax
import jax.numpy as jnp
import numpy as np
from jax import lax
from jax.experimental import pallas as pl
from jax.experimental.pallas import tpu as pltpu
from jax.experimental.pallas import tpu_sc as plsc

N = 10000
NP = 10240
E = 320000
F_IN = 128
NH = 64
LW = 16
NC = 2
NS = 16
EPC = E // NC
EPT = EPC // NS
CH = 125
NCHUNK = EPT // CH
RPT = NP // NS
ZROWS = 128

_f32 = jnp.float32


def _tc_pre1(x, w1p, asrc_p, adst_p):
    blk = 1000

    def body(x_ref, w_ref, as_ref, ad_ref, h_ref, s_ref, d_ref, c_ref, mx_ref):
        i = pl.program_id(0)
        h = jnp.dot(x_ref[...], w_ref[...], preferred_element_type=_f32)
        h_ref[...] = h
        s = jnp.dot(h, as_ref[...], preferred_element_type=_f32)
        d = jnp.dot(h, ad_ref[...], preferred_element_type=_f32)
        s_ref[...] = s
        d_ref[...] = d
        m = jnp.concatenate([jnp.max(s, axis=0, keepdims=True),
                             jnp.max(d, axis=0, keepdims=True)], axis=0)

        @pl.when(i == 0)
        def _():
            mx_ref[...] = m

        @pl.when(i > 0)
        def _():
            mx_ref[...] = jnp.maximum(mx_ref[...], m)

        c_ref[...] = mx_ref[0:1] + mx_ref[1:2]

    return pl.pallas_call(
        body,
        grid=(N // blk,),
        in_specs=[
            pl.BlockSpec((blk, F_IN), lambda i: (i, 0)),
            pl.BlockSpec((F_IN, NH), lambda i: (0, 0)),
            pl.BlockSpec((NH, LW), lambda i: (0, 0)),
            pl.BlockSpec((NH, LW), lambda i: (0, 0)),
        ],
        out_specs=[
            pl.BlockSpec((blk, NH), lambda i: (i, 0)),
            pl.BlockSpec((blk, LW), lambda i: (i, 0)),
            pl.BlockSpec((blk, LW), lambda i: (i, 0)),
            pl.BlockSpec((1, LW), lambda i: (0, 0)),
        ],
        out_shape=[
            jax.ShapeDtypeStruct((N, NH), _f32),
            jax.ShapeDtypeStruct((N, LW), _f32),
            jax.ShapeDtypeStruct((N, LW), _f32),
            jax.ShapeDtypeStruct((1, LW), _f32),
        ],
        scratch_shapes=[pltpu.VMEM((2, LW), _f32)],
    )(x, w1p, asrc_p, adst_p)


def _tc_mid(out1p, den1p, b1p, g1p, be1p, w2p, as2p, ad2p):
    blk = 1024
    ibn = 1.0 / np.sqrt(1.0 + 1e-5)

    def body(o_ref, dp_ref, b_ref, g_ref, be_ref, w_ref, as_ref, ad_ref,
             h_ref, s_ref, d_ref, c_ref, mx_ref):
        i = pl.program_id(0)
        rec = 1.0 / (dp_ref[0] + dp_ref[1] + 1e-16)
        rec_rep = jnp.concatenate([rec[:, 0:8]] * 8, axis=1)
        z = (o_ref[0] + o_ref[1]) * rec_rep + b_ref[...]
        z = jax.nn.relu(z * ibn * g_ref[...] + be_ref[...])
        h = jnp.dot(z, w_ref[...], preferred_element_type=_f32)
        h_ref[...] = h
        s = jnp.dot(h, as_ref[...], preferred_element_type=_f32)
        d = jnp.dot(h, ad_ref[...], preferred_element_type=_f32)
        s_ref[...] = s
        d_ref[...] = d
        m = jnp.concatenate([jnp.max(s, axis=0, keepdims=True),
                             jnp.max(d, axis=0, keepdims=True)], axis=0)

        @pl.when(i == 0)
        def _():
            mx_ref[...] = m

        @pl.when(i > 0)
        def _():
            mx_ref[...] = jnp.maximum(mx_ref[...], m)

        c_ref[...] = mx_ref[0:1] + mx_ref[1:2]

    return pl.pallas_call(
        body,
        grid=(NP // blk,),
        in_specs=[
            pl.BlockSpec((NC, blk, NH), lambda i: (0, i, 0)),
            pl.BlockSpec((NC, blk, LW), lambda i: (0, i, 0)),
            pl.BlockSpec((1, NH), lambda i: (0, 0)),
            pl.BlockSpec((1, NH), lambda i: (0, 0)),
            pl.BlockSpec((1, NH), lambda i: (0, 0)),
            pl.BlockSpec((NH, NH), lambda i: (0, 0)),
            pl.BlockSpec((NH, LW), lambda i: (0, 0)),
            pl.BlockSpec((NH, LW), lambda i: (0, 0)),
        ],
        out_specs=[
            pl.BlockSpec((blk, NH), lambda i: (i, 0)),
            pl.BlockSpec((blk, LW), lambda i: (i, 0)),
            pl.BlockSpec((blk, LW), lambda i: (i, 0)),
            pl.BlockSpec((1, LW), lambda i: (0, 0)),
        ],
        out_shape=[
            jax.ShapeDtypeStruct((NP, NH), _f32),
            jax.ShapeDtypeStruct((NP, LW), _f32),
            jax.ShapeDtypeStruct((NP, LW), _f32),
            jax.ShapeDtypeStruct((1, LW), _f32),
        ],
        scratch_shapes=[pltpu.VMEM((2, LW), _f32)],
    )(out1p, den1p, b1p, g1p, be1p, w2p, as2p, ad2p)


def _tc_head(out2p, den2p, b2, cg, cb, wl1, bl1, g1, be1, wl2, bl2, g2, be2,
             wf, bf):
    blk = 1024
    ibn = 1.0 / np.sqrt(1.0 + 1e-5)

    def body(o_ref, dp_ref, b2_ref, cg_ref, cb_ref, w1_ref, b1_ref, g1_ref,
             be1_ref, w2_ref, b2b_ref, g2_ref, be2_ref, wf_ref, bf_ref, y_ref):
        rec = 1.0 / (dp_ref[0, :, 0:1] + dp_ref[1, :, 0:1] + 1e-16)
        g = (o_ref[0] + o_ref[1]) * rec + b2_ref[...]
        g = g * ibn * cg_ref[...] + cb_ref[...]
        t = jnp.dot(g, w1_ref[...], preferred_element_type=_f32) + b1_ref[...]
        t = jax.nn.relu(t * ibn * g1_ref[...] + be1_ref[...])
        t = jnp.dot(t, w2_ref[...], preferred_element_type=_f32) + b2b_ref[...]
        t = jax.nn.relu(t * ibn * g2_ref[...] + be2_ref[...])
        y = jnp.dot(t, wf_ref[...], preferred_element_type=_f32) + bf_ref[...]
        y_ref[...] = jax.nn.sigmoid(y)

    vec = lambda: pl.BlockSpec((1, NH), lambda i: (0, 0))
    return pl.pallas_call(
        body,
        grid=(NP // blk,),
        in_specs=[
            pl.BlockSpec((NC, blk, NH), lambda i: (0, i, 0)),
            pl.BlockSpec((NC, blk, LW), lambda i: (0, i, 0)),
            vec(), vec(), vec(),
            pl.BlockSpec((NH, NH), lambda i: (0, 0)),
            vec(), vec(), vec(),
            pl.BlockSpec((NH, NH), lambda i: (0, 0)),
            vec(), vec(), vec(),
            pl.BlockSpec((NH, 1), lambda i: (0, 0)),
            pl.BlockSpec((1, 1), lambda i: (0, 0)),
        ],
        out_specs=pl.BlockSpec((blk, 1), lambda i: (i, 0)),
        out_shape=jax.ShapeDtypeStruct((NP, 1), _f32),
    )(out2p, den2p, b2, cg, cb, wl1, bl1, g1, be1, wl2, bl2, g2, be2, wf, bf)


def _sc_layer(src2d, dst2d, s_tab, d_tab, cvec_hbm, h_tab, heads8):
    @functools.partial(
        pl.kernel,
        mesh=plsc.VectorSubcoreMesh(core_axis_name="c", subcore_axis_name="s"),
        compiler_params=pltpu.CompilerParams(use_tc_tiling_on_sc=False),
        out_type=[
            jax.ShapeDtypeStruct((NC, NP, LW), _f32),
            jax.ShapeDtypeStruct((NC, NP, NH), _f32),
        ],
        scratch_types=[
            pltpu.VMEM((NCHUNK, CH), jnp.int32),
            pltpu.VMEM((NCHUNK, CH), jnp.int32),
            pltpu.VMEM((CH, LW), _f32),
            pltpu.VMEM((CH, LW), _f32),
            pltpu.VMEM((CH, LW), _f32),
            pltpu.VMEM((CH, LW), _f32),
            pltpu.VMEM((CH, NH), _f32),
            pltpu.VMEM((CH, NH), _f32),
            pltpu.VMEM((CH, LW), _f32),
            pltpu.VMEM((CH, LW), _f32),
            pltpu.VMEM((CH, NH), _f32),
            pltpu.VMEM((CH, NH), _f32),
            pltpu.VMEM((LW,), _f32),
            pltpu.VMEM((ZROWS, LW), _f32),
            pltpu.VMEM((ZROWS, NH), _f32),
            pltpu.VMEM_SHARED((NP, LW), _f32),
            pltpu.VMEM_SHARED((NP, NH), _f32),
            pltpu.SemaphoreType.DMA,
            pltpu.SemaphoreType.DMA,
            pltpu.SemaphoreType.DMA,
            pltpu.SemaphoreType.DMA,
        ],
    )
    def k(src_hbm, dst_hbm, s_hbm, d_hbm, c_hbm, h_hbm, den_hbm, out_hbm,
          sidx, didx, srow0, srow1, drow0, drow1, hrow0, hrow1,
          exb0, exb1, msg0, msg1, cvec, zb16, zb64, den_sh, out_sh,
          semg0, semg1, semw0, semw1):
        cid = lax.axis_index("c")
        sid = lax.axis_index("s")
        srow = (srow0, srow1)
        drow = (drow0, drow1)
        hrow = (hrow0, hrow1)
        exb = (exb0, exb1)
        msg = (msg0, msg1)
        semg = (semg0, semg1)
        semw = (semw0, semw1)

        @pl.loop(0, ZROWS)
        def _(r):
            zb16[r, :] = jnp.zeros((LW,), _f32)
            for j in range(NH // LW):
                zb64[r, pl.ds(j * LW, LW)] = jnp.zeros((LW,), _f32)

        @pl.loop(0, RPT // ZROWS)
        def _(j):
            pltpu.sync_copy(zb16, den_sh.at[pl.ds(sid * RPT + j * ZROWS, ZROWS)])
            pltpu.sync_copy(zb64, out_sh.at[pl.ds(sid * RPT + j * ZROWS, ZROWS)])

        w0 = (cid * NS + sid) * NCHUNK
        pltpu.sync_copy(src_hbm.at[pl.ds(w0, NCHUNK)], sidx)
        pltpu.sync_copy(dst_hbm.at[pl.ds(w0, NCHUNK)], didx)
        pltpu.sync_copy(c_hbm, cvec)
        plsc.subcore_barrier()
        cv = cvec[...]
        iot = lax.iota(jnp.int32, LW)
        repidx = (iot % 8 if heads8 else iot * 0).reshape(LW, 1)
        dnums = lax.GatherDimensionNumbers(
            offset_dims=(), collapsed_slice_dims=(0,), start_index_map=(0,))

        def start_gathers(i, b):
            pltpu.async_copy(s_hbm.at[sidx.at[i]], srow[b], semg[b])
            pltpu.async_copy(d_hbm.at[didx.at[i]], drow[b], semg[b])
            pltpu.async_copy(h_hbm.at[sidx.at[i]], hrow[b], semg[b])

        def wait_gathers(i, b):
            pltpu.make_async_copy(s_hbm.at[sidx.at[i]], srow[b], semg[b]).wait()
            pltpu.make_async_copy(d_hbm.at[didx.at[i]], drow[b], semg[b]).wait()
            pltpu.make_async_copy(h_hbm.at[sidx.at[i]], hrow[b], semg[b]).wait()

        def start_writes(i, b):
            pltpu.async_copy(exb[b], den_sh.at[didx.at[i]], semw[b], add=True)
            pltpu.async_copy(msg[b], out_sh.at[didx.at[i]], semw[b], add=True)

        def wait_writes(i, b):
            pltpu.make_async_copy(exb[b], den_sh.at[didx.at[i]], semw[b]).wait()
            pltpu.make_async_copy(msg[b], out_sh.at[didx.at[i]], semw[b]).wait()

        start_gathers(0, 0)
        start_gathers(1, 1)

        @pl.loop(0, NCHUNK // 2)
        def _(t):
            for b in range(2):
                i = t * 2 + b
                wait_gathers(i, b)

                @pl.when(i >= 2)
                def _():
                    wait_writes(i - 2, b)

                @pl.loop(0, CH)
                def _(r):
                    v = srow[b][r, :] + drow[b][r, :]
                    a = jnp.where(v >= 0.0, v, 0.2 * v)
                    e = jnp.exp(a - cv)
                    exb[b][r, :] = e
                    rep = lax.gather(e, repidx, dnums, (1,),
                                     mode=lax.GatherScatterMode.PROMISE_IN_BOUNDS)
                    for j in range(NH // LW):
                        msg[b][r, pl.ds(j * LW, LW)] = (
                            hrow[b][r, pl.ds(j * LW, LW)] * rep)

                start_writes(i, b)

                @pl.when(i + 2 < NCHUNK)
                def _():
                    start_gathers(i + 2, b)

        wait_writes(NCHUNK - 2, 0)
        wait_writes(NCHUNK - 1, 1)
        plsc.subcore_barrier()
        r0 = sid * RPT
        pltpu.sync_copy(den_sh.at[pl.ds(r0, RPT)],
                        den_hbm.at[cid].at[pl.ds(r0, RPT)])
        pltpu.sync_copy(out_sh.at[pl.ds(r0, RPT)],
                        out_hbm.at[cid].at[pl.ds(r0, RPT)])

    return k(src2d, dst2d, s_tab, d_tab, cvec_hbm, h_tab)


def kernel(x, edge_index, W1, a_src1, a_dst1, b1, W2, a_src2, a_dst2, b2,
           bn_c1_g, bn_c1_b, bn_c2_g, bn_c2_b, Wl1, bl1, bn1_g, bn1_b,
           Wl2, bl2, bn2_g, bn2_b, Wf, bf):
    src2d = edge_index[0].reshape(E // CH, CH)
    dst2d = edge_index[1].reshape(E // CH, CH)

    perm = np.array([(j % 8) * 8 + j // 8 for j in range(NH)])
    w1p = W1[:, perm]
    eye8 = jnp.eye(8, dtype=_f32)
    asrc_p = jnp.concatenate(
        [(a_src1.T[:, :, None] * eye8[None]).reshape(NH, 8),
         jnp.zeros((NH, 8), _f32)], axis=1)
    adst_p = jnp.concatenate(
        [(a_dst1.T[:, :, None] * eye8[None]).reshape(NH, 8),
         jnp.zeros((NH, 8), _f32)], axis=1)

    h1p, s1, d1, c1 = _tc_pre1(x, w1p, asrc_p, adst_p)
    den1p, out1p = _sc_layer(src2d, dst2d, s1, d1, c1.reshape(LW), h1p,
                             heads8=True)

    w2p = W2[perm, :]
    as2p = jnp.concatenate([a_src2.T, jnp.zeros((NH, LW - 1), _f32)], axis=1)
    ad2p = jnp.concatenate([a_dst2.T, jnp.zeros((NH, LW - 1), _f32)], axis=1)
    h2, s2, d2, c2 = _tc_mid(
        out1p, den1p, b1[perm].reshape(1, NH), bn_c1_g[perm].reshape(1, NH),
        bn_c1_b[perm].reshape(1, NH), w2p, as2p, ad2p)
    den2p, out2p = _sc_layer(src2d, dst2d, s2, d2, c2.reshape(LW), h2,
                             heads8=False)

    y = _tc_head(
        out2p, den2p, b2.reshape(1, NH), bn_c2_g.reshape(1, NH),
        bn_c2_b.reshape(1, NH), Wl1, bl1.reshape(1, NH),
        bn1_g.reshape(1, NH), bn1_b.reshape(1, NH), Wl2,
        bl2.reshape(1, NH), bn2_g.reshape(1, NH), bn2_b.reshape(1, NH),
        Wf, bf.reshape(1, 1))
    return y[:N]
